# Initial kernel scaffold; baseline (speedup 1.0000x reference)
import jax
import jax.numpy as jnp
from jax.experimental import pallas as pl


def kernel(x, edge_index, batch, W_gcn, b_gcn, W_fc, b_fc, W_a, b_a, W_c, b_c):
    raise NotImplementedError("write your pallas kernel here")


# speedup vs baseline: 25.8310x; 25.8310x over previous
import functools

import jax
import jax.numpy as jnp
from jax import lax
from jax.experimental import pallas as pl
from jax.experimental.pallas import tpu as pltpu
from jax.experimental.pallas import tpu_sc as plsc

N = 10000
E = 320000
D = 128
H = 128
F2 = 64
T = 16
G = 64

NC = 2
NS = 16
NW = NC * NS
EPW = E // NW
CH = 80
NCH = EPW // CH
ACC_ROWS = 10112
RPT = ACC_ROWS // NS


def _fill_f32(ref, rows, cols, value):
    cchunks = cols // 16
    v = jnp.full((16,), value, jnp.float32)

    def body(k, _):
        r = k // cchunks
        c = (k % cchunks) * 16
        ref[r, pl.ds(c, 16)] = v
        return 0

    lax.fori_loop(0, rows * cchunks, body, 0)


OFFS = tuple(range(0, RPT - CH, CH)) + (RPT - CH,)


def _sc_degree(dst3, idx3):
    mesh = plsc.VectorSubcoreMesh(core_axis_name="c", subcore_axis_name="s")

    @functools.partial(
        pl.kernel,
        out_type=jax.ShapeDtypeStruct((NC, ACC_ROWS, 16), jnp.float32),
        mesh=mesh,
        scratch_types=[
            pltpu.VMEM((NCH, CH), jnp.int32),
            pltpu.VMEM((len(OFFS), CH), jnp.int32),
            pltpu.VMEM((CH, 16), jnp.float32),
            pltpu.VMEM((CH, 16), jnp.float32),
            pltpu.VMEM((CH, 16), jnp.float32),
            pltpu.VMEM_SHARED((ACC_ROWS, 16), jnp.float32),
            pltpu.SemaphoreType.DMA,
        ],
    )
    def deg_kernel(dst_hbm, idx_hbm, cnt_hbm, dstv, idxv, onesv, zerov, obuf,
                   acc, sem):
        c = lax.axis_index("c")
        s = lax.axis_index("s")
        wid = s * NC + c

        _fill_f32(onesv, CH, 16, 1.0)
        _fill_f32(zerov, CH, 16, 0.0)
        pltpu.sync_copy(dst_hbm.at[wid], dstv)
        pltpu.sync_copy(idx_hbm.at[s], idxv)

        for b in range(len(OFFS)):
            pltpu.sync_copy(zerov, acc.at[idxv.at[b]])
        plsc.subcore_barrier()

        def chunk(j, _):
            pltpu.sync_copy(onesv, acc.at[dstv.at[j]], add=True)
            return 0

        lax.fori_loop(0, NCH, chunk, 0)
        plsc.subcore_barrier()

        for b in range(len(OFFS)):
            pltpu.async_copy(acc.at[idxv.at[b]], obuf, sem).wait()
            pltpu.sync_copy(obuf, cnt_hbm.at[c, pl.ds(s * RPT + OFFS[b], CH)])

    return deg_kernel(dst3, idx3)


def _sc_edges(hp, src3, dst3, idx3):
    mesh = plsc.VectorSubcoreMesh(core_axis_name="c", subcore_axis_name="s")

    @functools.partial(
        pl.kernel,
        out_type=jax.ShapeDtypeStruct((NC, ACC_ROWS, D), jnp.float32),
        mesh=mesh,
        scratch_types=[
            pltpu.VMEM((NCH, CH), jnp.int32),
            pltpu.VMEM((NCH, CH), jnp.int32),
            pltpu.VMEM((len(OFFS), CH), jnp.int32),
            pltpu.VMEM((CH, D), jnp.float32),
            pltpu.VMEM_SHARED((ACC_ROWS, D), jnp.float32),
            pltpu.SemaphoreType.DMA,
        ],
    )
    def edge_kernel(hp_hbm, src_hbm, dst_hbm, idx_hbm, part_hbm,
                    srcv, dstv, idxv, rows, acc, sem):
        c = lax.axis_index("c")
        s = lax.axis_index("s")
        wid = s * NC + c

        _fill_f32(rows, CH, D, 0.0)
        pltpu.sync_copy(src_hbm.at[wid], srcv)
        pltpu.sync_copy(dst_hbm.at[wid], dstv)
        pltpu.sync_copy(idx_hbm.at[s], idxv)

        for b in range(len(OFFS)):
            pltpu.sync_copy(rows, acc.at[idxv.at[b]])
        plsc.subcore_barrier()

        def chunk(j, _):
            pltpu.async_copy(hp_hbm.at[srcv.at[j]], rows, sem).wait()
            pltpu.sync_copy(rows, acc.at[dstv.at[j]], add=True)
            return 0

        lax.fori_loop(0, NCH, chunk, 0)
        plsc.subcore_barrier()

        for b in range(len(OFFS)):
            pltpu.async_copy(acc.at[idxv.at[b]], rows, sem).wait()
            pltpu.sync_copy(rows, part_hbm.at[c, pl.ds(s * RPT + OFFS[b], CH)])

    return edge_kernel(hp, src3, dst3, idx3)


def _tc_project(x, W_gcn, cnt):
    def body(x_ref, w_ref, cnt_ref, out_ref):
        deg = cnt_ref[0, :, 0:1] + cnt_ref[1, :, 0:1] + 1.0
        dinv = lax.rsqrt(deg)
        h = jnp.dot(x_ref[...], w_ref[...], preferred_element_type=jnp.float32)
        out_ref[...] = h * dinv

    return pl.pallas_call(
        body,
        out_shape=jax.ShapeDtypeStruct((N, D), jnp.float32),
    )(x, W_gcn, cnt)


def _tc_heads(part, hp, cnt, batch2d, b_gcn2, W_fc, b_fc2, W_a, b_a2, W_c, b_c2):
    def body(part_ref, hp_ref, cnt_ref, batch_ref, bg_ref, wfc_ref, bfc_ref,
             wa_ref, ba_ref, wc_ref, bc_ref, probs_ref, val_ref):
        deg = cnt_ref[0, :, 0:1] + cnt_ref[1, :, 0:1] + 1.0
        dinv = lax.rsqrt(deg)
        ssum = part_ref[0] + part_ref[1] + hp_ref[...]
        hr = jnp.maximum(ssum * dinv + bg_ref[...], 0.0)

        ids = batch_ref[...]
        gids = lax.broadcasted_iota(jnp.int32, (G, 1), 0)
        mask = (ids == gids).astype(jnp.float32)
        sums = jnp.dot(mask, hr, preferred_element_type=jnp.float32)
        cnts = jnp.sum(mask, axis=1, keepdims=True)
        gs = sums / jnp.maximum(cnts, 1.0)

        z = jnp.maximum(
            jnp.dot(gs, wfc_ref[...], preferred_element_type=jnp.float32)
            + bfc_ref[...], 0.0)
        logits = jnp.dot(z, wa_ref[...], preferred_element_type=jnp.float32) \
            + ba_ref[...]
        m = jnp.max(logits, axis=1, keepdims=True)
        e = jnp.exp(logits - m)
        probs_ref[...] = e / jnp.sum(e, axis=1, keepdims=True)
        val_ref[...] = jnp.dot(z, wc_ref[...],
                               preferred_element_type=jnp.float32) + bc_ref[...]

    return pl.pallas_call(
        body,
        out_shape=[
            jax.ShapeDtypeStruct((G, T), jnp.float32),
            jax.ShapeDtypeStruct((G, 1), jnp.float32),
        ],
    )(part, hp, cnt, batch2d, b_gcn2, W_fc, b_fc2, W_a, b_a2, W_c, b_c2)


def kernel(x, edge_index, batch, W_gcn, b_gcn, W_fc, b_fc, W_a, b_a, W_c, b_c):
    src3 = edge_index[0].reshape(NW, NCH, CH)
    dst3 = edge_index[1].reshape(NW, NCH, CH)
    idx3 = (jnp.arange(NS, dtype=jnp.int32)[:, None, None] * RPT
            + jnp.array(OFFS, jnp.int32)[None, :, None]
            + jnp.arange(CH, dtype=jnp.int32)[None, None, :])
    cnt = _sc_degree(dst3, idx3)[:, :N, :]
    hp = _tc_project(x, W_gcn, cnt)
    part = _sc_edges(hp, src3, dst3, idx3)[:, :N, :]
    probs, val = _tc_heads(
        part, hp, cnt,
        batch.reshape(1, N),
        b_gcn.reshape(1, H), W_fc, b_fc.reshape(1, F2),
        W_a, b_a.reshape(1, T), W_c, b_c.reshape(1, 1),
    )
    return (probs, val)
